# baseline (speedup 1.0000x reference)
import functools
import math

import jax
import jax.numpy as jnp
from jax import lax
from jax.experimental import pallas as pl
from jax.experimental.pallas import tpu as pltpu
from jax.experimental.pallas import tpu_sc as plsc

_TI = 256
_NC = 2
_NS = 16
_NW = _NC * _NS

_PAIRS = [(0, 1), (0, 2), (0, 3), (1, 2), (1, 3), (2, 3)]
_N_SWEEPS = 5


_NCHUNK = 4


def _stats_kernel(se_ref, te_ref, xp_ref, temp_ref, w_ref, c_ref, u_ref):
    ti = _TI
    n = te_ref.shape[2]
    nc = n // _NCHUNK
    se = se_ref[0]
    sqd = math.sqrt(se_ref.shape[1])
    temp = temp_ref[0, 0, 0]

    m = None
    cor = None
    ssum = None
    for ci in range(_NCHUNK):
        te = te_ref[0, :, pl.ds(ci * nc, nc)]
        s = lax.dot_general(se, te, (((0,), (0,)), ((), ())),
                            preferred_element_type=jnp.float32)
        sc = temp * (s / sqd)
        cm = jnp.max(sc, axis=1, keepdims=True)
        iota = lax.broadcasted_iota(jnp.int32, (ti, nc), 1) + ci * nc
        ca = jnp.min(jnp.where(sc == cm, iota, n), axis=1, keepdims=True)
        if ci == 0:
            m = cm
            cor = ca
            ssum = jnp.sum(jnp.exp(sc - m), axis=1, keepdims=True)
        else:
            m_new = jnp.maximum(m, cm)
            cor = jnp.where(cm > m, ca, cor)
            ssum = (ssum * jnp.exp(m - m_new)
                    + jnp.sum(jnp.exp(sc - m_new), axis=1, keepdims=True))
            m = m_new
    w = 1.0 / ssum
    w_ref[0, 0] = w
    c_ref[0, 0] = cor
    u_ref[0, 0] = w * xp_ref[0]


def _make_sc_reduce(B, N):
    cpb = _NW // B
    rpw = N // cpb
    steps = rpw // 16

    mesh = plsc.VectorSubcoreMesh(core_axis_name="c", subcore_axis_name="s",
                                  num_cores=_NC, num_subcores=_NS)

    @functools.partial(
        pl.kernel, mesh=mesh,
        out_type=jax.ShapeDtypeStruct((_NW, 1, 16), jnp.float32),
        scratch_types=[
            pltpu.VMEM((rpw,), jnp.int32),
            pltpu.VMEM((rpw, 16), jnp.float32),
            pltpu.VMEM((128, 128), jnp.float32),
            pltpu.VMEM((128, 128), jnp.float32),
            pltpu.VMEM((1, 16), jnp.float32),
            pltpu.SemaphoreType.DMA,
            pltpu.SemaphoreType.DMA,
        ],
    )
    def sc_reduce(tgt16_hbm, cor_hbm, u_hbm, out_hbm,
                  idx_v, uv, yv0, yv1, outv, sem0, sem1):
        wid = lax.axis_index("s") * _NC + lax.axis_index("c")
        b = wid // cpb
        base = (wid % cpb) * rpw
        pltpu.sync_copy(cor_hbm.at[b, pl.ds(base, rpw)], idx_v)
        pltpu.sync_copy(u_hbm.at[b, pl.ds(base, rpw)], uv)

        nch = rpw // 128
        bufs = [yv0, yv1]
        sems = [sem0, sem1]

        def fire(ch):
            return pltpu.async_copy(
                tgt16_hbm.at[b].at[idx_v.at[pl.ds(ch * 128, 128)]],
                bufs[ch % 2], sems[ch % 2])

        accs = [jnp.zeros((16,), jnp.float32) for _ in range(4)]
        pending = fire(0)
        for ch in range(nch):
            nxt = fire(ch + 1) if ch + 1 < nch else None
            pending.wait()
            yv = bufs[ch % 2]
            for r in range(0, 128, 4):
                for j in range(4):
                    accs[j] = (accs[j] + uv[ch * 128 + r + j, :]
                               * yv[r + j, pl.ds(0, 16)])
            pending = nxt
        outv[0, :] = (accs[0] + accs[1]) + (accs[2] + accs[3])
        pltpu.sync_copy(outv, out_hbm.at[wid])

    return sc_reduce


def _solve_kernel(parts_ref, r_ref, t_ref):
    eps = 1e-07
    pv = parts_ref[...]
    vsum = jnp.sum(pv, axis=1, keepdims=True)

    def e(j):
        return vsum[:, :, j:j + 1]

    sw = e(0)
    den = sw + eps
    c = sw / den
    mux_c = jnp.concatenate([e(4), e(8), e(12)], 1) / den
    muy_r = jnp.concatenate([e(1), e(2), e(3)], 2) / den
    muy_c = jnp.concatenate([e(1), e(2), e(3)], 1) / den
    rows = [jnp.concatenate([e(4 * i + 5), e(4 * i + 6), e(4 * i + 7)], 2)
            for i in range(3)]
    araw = jnp.concatenate(rows, 1)
    a = araw / den - (2.0 - c) * (mux_c * muy_r)

    def ent(i, j):
        return a[:, i:i + 1, j:j + 1]

    s00, s01, s02 = ent(0, 0), ent(0, 1), ent(0, 2)
    s10, s11, s12 = ent(1, 0), ent(1, 1), ent(1, 2)
    s20, s21, s22 = ent(2, 0), ent(2, 1), ent(2, 2)

    km = {}
    km[(0, 0)] = s00 + s11 + s22
    km[(0, 1)] = s12 - s21
    km[(0, 2)] = s20 - s02
    km[(0, 3)] = s01 - s10
    km[(1, 1)] = s00 - s11 - s22
    km[(1, 2)] = s01 + s10
    km[(1, 3)] = s20 + s02
    km[(2, 2)] = -s00 + s11 - s22
    km[(2, 3)] = s12 + s21
    km[(3, 3)] = -s00 - s11 + s22

    def kget(i, j):
        return km[(i, j)] if i <= j else km[(j, i)]

    def kset(i, j, val):
        km[(min(i, j), max(i, j))] = val

    one = jnp.ones_like(s00)
    zero = jnp.zeros_like(s00)
    vm = {(i, j): (one if i == j else zero)
          for i in range(4) for j in range(4)}

    for _ in range(_N_SWEEPS):
        for (p, q) in _PAIRS:
            apq = kget(p, q)
            diff = kget(q, q) - kget(p, p)
            sn = jnp.where(diff >= 0.0, 1.0, -1.0)
            den = jnp.abs(diff) + jnp.sqrt(diff * diff + 4.0 * apq * apq) + 1e-38
            tj = 2.0 * sn * apq / den
            cc = lax.rsqrt(tj * tj + 1.0)
            ss = tj * cc
            kpp, kqq, kpq = kget(p, p), kget(q, q), kget(p, q)
            kset(p, p, cc * cc * kpp - 2.0 * ss * cc * kpq + ss * ss * kqq)
            kset(q, q, ss * ss * kpp + 2.0 * ss * cc * kpq + cc * cc * kqq)
            kset(p, q, zero)
            for r in range(4):
                if r == p or r == q:
                    continue
                kpr, kqr = kget(p, r), kget(q, r)
                kset(p, r, cc * kpr - ss * kqr)
                kset(q, r, ss * kpr + cc * kqr)
            for i in range(4):
                vip, viq = vm[(i, p)], vm[(i, q)]
                vm[(i, p)] = cc * vip - ss * viq
                vm[(i, q)] = ss * vip + cc * viq

    d0, d1, d2, d3 = kget(0, 0), kget(1, 1), kget(2, 2), kget(3, 3)
    dmax = jnp.maximum(jnp.maximum(d0, d1), jnp.maximum(d2, d3))

    def pick(i):
        return jnp.where(d0 == dmax, vm[(i, 0)],
                         jnp.where(d1 == dmax, vm[(i, 1)],
                                   jnp.where(d2 == dmax, vm[(i, 2)],
                                             vm[(i, 3)])))

    q0, qx, qy, qz = pick(0), pick(1), pick(2), pick(3)
    qn = 1.0 / jnp.sqrt(q0 * q0 + qx * qx + qy * qy + qz * qz)
    q0, qx, qy, qz = q0 * qn, qx * qn, qy * qn, qz * qn
    r0 = jnp.concatenate([q0 * q0 + qx * qx - qy * qy - qz * qz,
                          2.0 * (qx * qy - q0 * qz),
                          2.0 * (qx * qz + q0 * qy)], 2)
    r1 = jnp.concatenate([2.0 * (qy * qx + q0 * qz),
                          q0 * q0 - qx * qx + qy * qy - qz * qz,
                          2.0 * (qy * qz - q0 * qx)], 2)
    r2 = jnp.concatenate([2.0 * (qz * qx - q0 * qy),
                          2.0 * (qz * qy + q0 * qx),
                          q0 * q0 - qx * qx - qy * qy + qz * qz], 2)
    rmat = jnp.concatenate([r0, r1, r2], 1)
    r_ref[...] = rmat

    rmux = rmat[:, :, 0:1] * mux_c[:, 0:1, :]
    for bidx in range(1, 3):
        rmux = rmux + rmat[:, :, bidx:bidx + 1] * mux_c[:, bidx:bidx + 1, :]
    t_ref[...] = muy_c - rmux


def kernel(src_embedding, tgt_embedding, src, tgt, temperature, is_corr):
    B, d, N = src_embedding.shape
    ti = _TI
    ni = N // ti
    cpb = _NW // B
    temp3 = temperature.reshape(B, 1, 1)

    src_t = jnp.transpose(src, (0, 2, 1))
    xp = jnp.repeat(
        jnp.concatenate([jnp.ones((B, N, 1), jnp.float32), src_t], axis=2),
        4, axis=2)
    weight4, corres4, u16 = pl.pallas_call(
        _stats_kernel,
        grid=(B, ni),
        in_specs=[
            pl.BlockSpec((1, d, ti), lambda b, i: (b, 0, i)),
            pl.BlockSpec((1, d, N), lambda b, i: (b, 0, 0)),
            pl.BlockSpec((1, ti, 16), lambda b, i: (b, i, 0)),
            pl.BlockSpec((1, 1, 1), lambda b, i: (b, 0, 0)),
        ],
        out_specs=[
            pl.BlockSpec((1, 1, ti, 1), lambda b, i: (b, i, 0, 0)),
            pl.BlockSpec((1, 1, ti, 1), lambda b, i: (b, i, 0, 0)),
            pl.BlockSpec((1, 1, ti, 16), lambda b, i: (b, i, 0, 0)),
        ],
        out_shape=[
            jax.ShapeDtypeStruct((B, ni, ti, 1), jnp.float32),
            jax.ShapeDtypeStruct((B, ni, ti, 1), jnp.int32),
            jax.ShapeDtypeStruct((B, ni, ti, 16), jnp.float32),
        ],
    )(src_embedding, tgt_embedding, xp, temp3)

    weight2 = weight4.reshape(B, N)
    corres2 = corres4.reshape(B, N)

    tgt_t = jnp.transpose(tgt, (0, 2, 1))
    v4 = jnp.concatenate([jnp.ones((B, N, 1), jnp.float32), tgt_t], axis=2)
    tgt128 = jnp.pad(jnp.tile(v4, (1, 1, 4)), ((0, 0), (0, 0), (0, 112)))

    parts = _make_sc_reduce(B, N)(tgt128, corres2, u16.reshape(B, N, 16))
    parts3 = parts.reshape(B, cpb, 16)

    rmat, tvec = pl.pallas_call(
        _solve_kernel,
        out_shape=[
            jax.ShapeDtypeStruct((B, 3, 3), jnp.float32),
            jax.ShapeDtypeStruct((B, 3, 1), jnp.float32),
        ],
    )(parts3)

    return (rmat, tvec.reshape(B, 3),
            corres2.reshape(B, N, 1), weight2.reshape(B, N, 1))


# speedup vs baseline: 1.1054x; 1.1054x over previous
import functools
import math

import jax
import jax.numpy as jnp
from jax import lax
from jax.experimental import pallas as pl
from jax.experimental.pallas import tpu as pltpu
from jax.experimental.pallas import tpu_sc as plsc

_TI = 512
_NC = 2
_NS = 16
_NW = _NC * _NS

_PAIRS = [(0, 1), (0, 2), (0, 3), (1, 2), (1, 3), (2, 3)]
_N_SWEEPS = 5


_NCHUNK = 2


def _stats_kernel(se_ref, te_ref, xp_ref, temp_ref, w_ref, c_ref, u_ref):
    ti = _TI
    n = te_ref.shape[2]
    nc = n // _NCHUNK
    se = se_ref[0]
    sqd = math.sqrt(se_ref.shape[1])
    temp = temp_ref[0, 0, 0]

    m = None
    cor = None
    ssum = None
    for ci in range(_NCHUNK):
        te = te_ref[0, :, pl.ds(ci * nc, nc)]
        s = lax.dot_general(se, te, (((0,), (0,)), ((), ())),
                            preferred_element_type=jnp.float32)
        sc = temp * (s / sqd)
        cm = jnp.max(sc, axis=1, keepdims=True)
        iota = lax.broadcasted_iota(jnp.int32, (ti, nc), 1) + ci * nc
        ca = jnp.min(jnp.where(sc == cm, iota, n), axis=1, keepdims=True)
        if ci == 0:
            m = cm
            cor = ca
            ssum = jnp.sum(jnp.exp(sc - m), axis=1, keepdims=True)
        else:
            m_new = jnp.maximum(m, cm)
            cor = jnp.where(cm > m, ca, cor)
            ssum = (ssum * jnp.exp(m - m_new)
                    + jnp.sum(jnp.exp(sc - m_new), axis=1, keepdims=True))
            m = m_new
    w = 1.0 / ssum
    w_ref[0, 0] = w
    c_ref[0, 0] = cor
    u_ref[0, 0] = w * xp_ref[0]


def _make_sc_reduce(B, N):
    cpb = _NW // B
    rpw = N // cpb
    steps = rpw // 16

    mesh = plsc.VectorSubcoreMesh(core_axis_name="c", subcore_axis_name="s",
                                  num_cores=_NC, num_subcores=_NS)

    @functools.partial(
        pl.kernel, mesh=mesh,
        out_type=jax.ShapeDtypeStruct((_NW, 1, 16), jnp.float32),
        scratch_types=[
            pltpu.VMEM((rpw,), jnp.int32),
            pltpu.VMEM((rpw, 16), jnp.float32),
            pltpu.VMEM((128, 128), jnp.float32),
            pltpu.VMEM((128, 128), jnp.float32),
            pltpu.VMEM((1, 16), jnp.float32),
            pltpu.SemaphoreType.DMA,
            pltpu.SemaphoreType.DMA,
        ],
    )
    def sc_reduce(tgt16_hbm, cor_hbm, u_hbm, out_hbm,
                  idx_v, uv, yv0, yv1, outv, sem0, sem1):
        wid = lax.axis_index("s") * _NC + lax.axis_index("c")
        b = wid // cpb
        base = (wid % cpb) * rpw
        pltpu.sync_copy(cor_hbm.at[b, pl.ds(base, rpw)], idx_v)
        pltpu.sync_copy(u_hbm.at[b, pl.ds(base, rpw)], uv)

        nch = rpw // 128
        bufs = [yv0, yv1]
        sems = [sem0, sem1]

        def fire(ch):
            return pltpu.async_copy(
                tgt16_hbm.at[b].at[idx_v.at[pl.ds(ch * 128, 128)]],
                bufs[ch % 2], sems[ch % 2])

        accs = [jnp.zeros((16,), jnp.float32) for _ in range(4)]
        pending = fire(0)
        for ch in range(nch):
            nxt = fire(ch + 1) if ch + 1 < nch else None
            pending.wait()
            yv = bufs[ch % 2]
            for r in range(0, 128, 4):
                for j in range(4):
                    accs[j] = (accs[j] + uv[ch * 128 + r + j, :]
                               * yv[r + j, pl.ds(0, 16)])
            pending = nxt
        outv[0, :] = (accs[0] + accs[1]) + (accs[2] + accs[3])
        pltpu.sync_copy(outv, out_hbm.at[wid])

    return sc_reduce


def _solve_kernel(parts_ref, r_ref, t_ref):
    eps = 1e-07
    pv = parts_ref[...]
    vsum = jnp.sum(pv, axis=1, keepdims=True)

    def e(j):
        return vsum[:, :, j:j + 1]

    sw = e(0)
    den = sw + eps
    c = sw / den
    mux_c = jnp.concatenate([e(4), e(8), e(12)], 1) / den
    muy_r = jnp.concatenate([e(1), e(2), e(3)], 2) / den
    muy_c = jnp.concatenate([e(1), e(2), e(3)], 1) / den
    rows = [jnp.concatenate([e(4 * i + 5), e(4 * i + 6), e(4 * i + 7)], 2)
            for i in range(3)]
    araw = jnp.concatenate(rows, 1)
    a = araw / den - (2.0 - c) * (mux_c * muy_r)

    def ent(i, j):
        return a[:, i:i + 1, j:j + 1]

    s00, s01, s02 = ent(0, 0), ent(0, 1), ent(0, 2)
    s10, s11, s12 = ent(1, 0), ent(1, 1), ent(1, 2)
    s20, s21, s22 = ent(2, 0), ent(2, 1), ent(2, 2)

    km = {}
    km[(0, 0)] = s00 + s11 + s22
    km[(0, 1)] = s12 - s21
    km[(0, 2)] = s20 - s02
    km[(0, 3)] = s01 - s10
    km[(1, 1)] = s00 - s11 - s22
    km[(1, 2)] = s01 + s10
    km[(1, 3)] = s20 + s02
    km[(2, 2)] = -s00 + s11 - s22
    km[(2, 3)] = s12 + s21
    km[(3, 3)] = -s00 - s11 + s22

    def kget(i, j):
        return km[(i, j)] if i <= j else km[(j, i)]

    def kset(i, j, val):
        km[(min(i, j), max(i, j))] = val

    one = jnp.ones_like(s00)
    zero = jnp.zeros_like(s00)
    vm = {(i, j): (one if i == j else zero)
          for i in range(4) for j in range(4)}

    for _ in range(_N_SWEEPS):
        for (p, q) in _PAIRS:
            apq = kget(p, q)
            diff = kget(q, q) - kget(p, p)
            sn = jnp.where(diff >= 0.0, 1.0, -1.0)
            den = jnp.abs(diff) + jnp.sqrt(diff * diff + 4.0 * apq * apq) + 1e-38
            tj = 2.0 * sn * apq / den
            cc = lax.rsqrt(tj * tj + 1.0)
            ss = tj * cc
            kpp, kqq, kpq = kget(p, p), kget(q, q), kget(p, q)
            kset(p, p, cc * cc * kpp - 2.0 * ss * cc * kpq + ss * ss * kqq)
            kset(q, q, ss * ss * kpp + 2.0 * ss * cc * kpq + cc * cc * kqq)
            kset(p, q, zero)
            for r in range(4):
                if r == p or r == q:
                    continue
                kpr, kqr = kget(p, r), kget(q, r)
                kset(p, r, cc * kpr - ss * kqr)
                kset(q, r, ss * kpr + cc * kqr)
            for i in range(4):
                vip, viq = vm[(i, p)], vm[(i, q)]
                vm[(i, p)] = cc * vip - ss * viq
                vm[(i, q)] = ss * vip + cc * viq

    d0, d1, d2, d3 = kget(0, 0), kget(1, 1), kget(2, 2), kget(3, 3)
    dmax = jnp.maximum(jnp.maximum(d0, d1), jnp.maximum(d2, d3))

    def pick(i):
        return jnp.where(d0 == dmax, vm[(i, 0)],
                         jnp.where(d1 == dmax, vm[(i, 1)],
                                   jnp.where(d2 == dmax, vm[(i, 2)],
                                             vm[(i, 3)])))

    q0, qx, qy, qz = pick(0), pick(1), pick(2), pick(3)
    qn = 1.0 / jnp.sqrt(q0 * q0 + qx * qx + qy * qy + qz * qz)
    q0, qx, qy, qz = q0 * qn, qx * qn, qy * qn, qz * qn
    r0 = jnp.concatenate([q0 * q0 + qx * qx - qy * qy - qz * qz,
                          2.0 * (qx * qy - q0 * qz),
                          2.0 * (qx * qz + q0 * qy)], 2)
    r1 = jnp.concatenate([2.0 * (qy * qx + q0 * qz),
                          q0 * q0 - qx * qx + qy * qy - qz * qz,
                          2.0 * (qy * qz - q0 * qx)], 2)
    r2 = jnp.concatenate([2.0 * (qz * qx - q0 * qy),
                          2.0 * (qz * qy + q0 * qx),
                          q0 * q0 - qx * qx - qy * qy + qz * qz], 2)
    rmat = jnp.concatenate([r0, r1, r2], 1)
    r_ref[...] = rmat

    rmux = rmat[:, :, 0:1] * mux_c[:, 0:1, :]
    for bidx in range(1, 3):
        rmux = rmux + rmat[:, :, bidx:bidx + 1] * mux_c[:, bidx:bidx + 1, :]
    t_ref[...] = muy_c - rmux


def kernel(src_embedding, tgt_embedding, src, tgt, temperature, is_corr):
    B, d, N = src_embedding.shape
    ti = _TI
    ni = N // ti
    cpb = _NW // B
    temp3 = temperature.reshape(B, 1, 1)

    src_t = jnp.transpose(src, (0, 2, 1))
    xp = jnp.repeat(
        jnp.concatenate([jnp.ones((B, N, 1), jnp.float32), src_t], axis=2),
        4, axis=2)
    weight4, corres4, u16 = pl.pallas_call(
        _stats_kernel,
        grid=(B, ni),
        in_specs=[
            pl.BlockSpec((1, d, ti), lambda b, i: (b, 0, i)),
            pl.BlockSpec((1, d, N), lambda b, i: (b, 0, 0)),
            pl.BlockSpec((1, ti, 16), lambda b, i: (b, i, 0)),
            pl.BlockSpec((1, 1, 1), lambda b, i: (b, 0, 0)),
        ],
        out_specs=[
            pl.BlockSpec((1, 1, ti, 1), lambda b, i: (b, i, 0, 0)),
            pl.BlockSpec((1, 1, ti, 1), lambda b, i: (b, i, 0, 0)),
            pl.BlockSpec((1, 1, ti, 16), lambda b, i: (b, i, 0, 0)),
        ],
        out_shape=[
            jax.ShapeDtypeStruct((B, ni, ti, 1), jnp.float32),
            jax.ShapeDtypeStruct((B, ni, ti, 1), jnp.int32),
            jax.ShapeDtypeStruct((B, ni, ti, 16), jnp.float32),
        ],
    )(src_embedding, tgt_embedding, xp, temp3)

    weight2 = weight4.reshape(B, N)
    corres2 = corres4.reshape(B, N)

    tgt_t = jnp.transpose(tgt, (0, 2, 1))
    v4 = jnp.concatenate([jnp.ones((B, N, 1), jnp.float32), tgt_t], axis=2)
    tgt128 = jnp.pad(jnp.tile(v4, (1, 1, 4)), ((0, 0), (0, 0), (0, 112)))

    parts = _make_sc_reduce(B, N)(tgt128, corres2, u16.reshape(B, N, 16))
    parts3 = parts.reshape(B, cpb, 16)

    rmat, tvec = pl.pallas_call(
        _solve_kernel,
        out_shape=[
            jax.ShapeDtypeStruct((B, 3, 3), jnp.float32),
            jax.ShapeDtypeStruct((B, 3, 1), jnp.float32),
        ],
    )(parts3)

    return (rmat, tvec.reshape(B, 3),
            corres2.reshape(B, N, 1), weight2.reshape(B, N, 1))


# speedup vs baseline: 1.1745x; 1.0625x over previous
import functools
import math

import jax
import jax.numpy as jnp
from jax import lax
from jax.experimental import pallas as pl
from jax.experimental.pallas import tpu as pltpu
from jax.experimental.pallas import tpu_sc as plsc

_TI = 1024
_NC = 2
_NS = 16
_NW = _NC * _NS

_PAIRS = [(0, 1), (0, 2), (0, 3), (1, 2), (1, 3), (2, 3)]
_N_SWEEPS = 5


_NCHUNK = 2


def _stats_kernel(se_ref, te_ref, xp_ref, temp_ref, w_ref, c_ref, u_ref):
    ti = _TI
    n = te_ref.shape[2]
    nc = n // _NCHUNK
    se = se_ref[0]
    sqd = math.sqrt(se_ref.shape[1])
    temp = temp_ref[0, 0, 0]

    m = None
    cor = None
    ssum = None
    for ci in range(_NCHUNK):
        te = te_ref[0, :, pl.ds(ci * nc, nc)]
        s = lax.dot_general(se, te, (((0,), (0,)), ((), ())),
                            preferred_element_type=jnp.float32)
        sc = temp * (s / sqd)
        cm = jnp.max(sc, axis=1, keepdims=True)
        iota = lax.broadcasted_iota(jnp.int32, (ti, nc), 1) + ci * nc
        ca = jnp.min(jnp.where(sc == cm, iota, n), axis=1, keepdims=True)
        if ci == 0:
            m = cm
            cor = ca
            ssum = jnp.sum(jnp.exp(sc - m), axis=1, keepdims=True)
        else:
            m_new = jnp.maximum(m, cm)
            cor = jnp.where(cm > m, ca, cor)
            ssum = (ssum * jnp.exp(m - m_new)
                    + jnp.sum(jnp.exp(sc - m_new), axis=1, keepdims=True))
            m = m_new
    w = 1.0 / ssum
    w_ref[0, 0] = w
    c_ref[0, 0] = cor
    u_ref[0, 0] = w * xp_ref[0]


def _make_sc_reduce(B, N):
    cpb = _NW // B
    rpw = N // cpb
    steps = rpw // 16

    mesh = plsc.VectorSubcoreMesh(core_axis_name="c", subcore_axis_name="s",
                                  num_cores=_NC, num_subcores=_NS)

    @functools.partial(
        pl.kernel, mesh=mesh,
        out_type=jax.ShapeDtypeStruct((_NW, 1, 16), jnp.float32),
        scratch_types=[
            pltpu.VMEM((rpw,), jnp.int32),
            pltpu.VMEM((rpw, 16), jnp.float32),
            pltpu.VMEM((128, 128), jnp.float32),
            pltpu.VMEM((128, 128), jnp.float32),
            pltpu.VMEM((1, 16), jnp.float32),
            pltpu.SemaphoreType.DMA,
            pltpu.SemaphoreType.DMA,
        ],
    )
    def sc_reduce(tgt16_hbm, cor_hbm, u_hbm, out_hbm,
                  idx_v, uv, yv0, yv1, outv, sem0, sem1):
        wid = lax.axis_index("s") * _NC + lax.axis_index("c")
        b = wid // cpb
        base = (wid % cpb) * rpw
        pltpu.sync_copy(cor_hbm.at[b, pl.ds(base, rpw)], idx_v)
        pltpu.sync_copy(u_hbm.at[b, pl.ds(base, rpw)], uv)

        nch = rpw // 128
        bufs = [yv0, yv1]
        sems = [sem0, sem1]

        def fire(ch):
            return pltpu.async_copy(
                tgt16_hbm.at[b].at[idx_v.at[pl.ds(ch * 128, 128)]],
                bufs[ch % 2], sems[ch % 2])

        accs = [jnp.zeros((16,), jnp.float32) for _ in range(4)]
        pending = fire(0)
        for ch in range(nch):
            nxt = fire(ch + 1) if ch + 1 < nch else None
            pending.wait()
            yv = bufs[ch % 2]
            for r in range(0, 128, 4):
                for j in range(4):
                    accs[j] = (accs[j] + uv[ch * 128 + r + j, :]
                               * yv[r + j, pl.ds(0, 16)])
            pending = nxt
        outv[0, :] = (accs[0] + accs[1]) + (accs[2] + accs[3])
        pltpu.sync_copy(outv, out_hbm.at[wid])

    return sc_reduce


def _solve_kernel(parts_ref, r_ref, t_ref):
    eps = 1e-07
    pv = parts_ref[...]
    vsum = jnp.sum(pv, axis=1, keepdims=True)

    def e(j):
        return vsum[:, :, j:j + 1]

    sw = e(0)
    den = sw + eps
    c = sw / den
    mux_c = jnp.concatenate([e(4), e(8), e(12)], 1) / den
    muy_r = jnp.concatenate([e(1), e(2), e(3)], 2) / den
    muy_c = jnp.concatenate([e(1), e(2), e(3)], 1) / den
    rows = [jnp.concatenate([e(4 * i + 5), e(4 * i + 6), e(4 * i + 7)], 2)
            for i in range(3)]
    araw = jnp.concatenate(rows, 1)
    a = araw / den - (2.0 - c) * (mux_c * muy_r)

    def ent(i, j):
        return a[:, i:i + 1, j:j + 1]

    s00, s01, s02 = ent(0, 0), ent(0, 1), ent(0, 2)
    s10, s11, s12 = ent(1, 0), ent(1, 1), ent(1, 2)
    s20, s21, s22 = ent(2, 0), ent(2, 1), ent(2, 2)

    km = {}
    km[(0, 0)] = s00 + s11 + s22
    km[(0, 1)] = s12 - s21
    km[(0, 2)] = s20 - s02
    km[(0, 3)] = s01 - s10
    km[(1, 1)] = s00 - s11 - s22
    km[(1, 2)] = s01 + s10
    km[(1, 3)] = s20 + s02
    km[(2, 2)] = -s00 + s11 - s22
    km[(2, 3)] = s12 + s21
    km[(3, 3)] = -s00 - s11 + s22

    def kget(i, j):
        return km[(i, j)] if i <= j else km[(j, i)]

    def kset(i, j, val):
        km[(min(i, j), max(i, j))] = val

    one = jnp.ones_like(s00)
    zero = jnp.zeros_like(s00)
    vm = {(i, j): (one if i == j else zero)
          for i in range(4) for j in range(4)}

    for _ in range(_N_SWEEPS):
        for (p, q) in _PAIRS:
            apq = kget(p, q)
            diff = kget(q, q) - kget(p, p)
            sn = jnp.where(diff >= 0.0, 1.0, -1.0)
            den = jnp.abs(diff) + jnp.sqrt(diff * diff + 4.0 * apq * apq) + 1e-38
            tj = 2.0 * sn * apq / den
            cc = lax.rsqrt(tj * tj + 1.0)
            ss = tj * cc
            kpp, kqq, kpq = kget(p, p), kget(q, q), kget(p, q)
            kset(p, p, cc * cc * kpp - 2.0 * ss * cc * kpq + ss * ss * kqq)
            kset(q, q, ss * ss * kpp + 2.0 * ss * cc * kpq + cc * cc * kqq)
            kset(p, q, zero)
            for r in range(4):
                if r == p or r == q:
                    continue
                kpr, kqr = kget(p, r), kget(q, r)
                kset(p, r, cc * kpr - ss * kqr)
                kset(q, r, ss * kpr + cc * kqr)
            for i in range(4):
                vip, viq = vm[(i, p)], vm[(i, q)]
                vm[(i, p)] = cc * vip - ss * viq
                vm[(i, q)] = ss * vip + cc * viq

    d0, d1, d2, d3 = kget(0, 0), kget(1, 1), kget(2, 2), kget(3, 3)
    dmax = jnp.maximum(jnp.maximum(d0, d1), jnp.maximum(d2, d3))

    def pick(i):
        return jnp.where(d0 == dmax, vm[(i, 0)],
                         jnp.where(d1 == dmax, vm[(i, 1)],
                                   jnp.where(d2 == dmax, vm[(i, 2)],
                                             vm[(i, 3)])))

    q0, qx, qy, qz = pick(0), pick(1), pick(2), pick(3)
    qn = 1.0 / jnp.sqrt(q0 * q0 + qx * qx + qy * qy + qz * qz)
    q0, qx, qy, qz = q0 * qn, qx * qn, qy * qn, qz * qn
    r0 = jnp.concatenate([q0 * q0 + qx * qx - qy * qy - qz * qz,
                          2.0 * (qx * qy - q0 * qz),
                          2.0 * (qx * qz + q0 * qy)], 2)
    r1 = jnp.concatenate([2.0 * (qy * qx + q0 * qz),
                          q0 * q0 - qx * qx + qy * qy - qz * qz,
                          2.0 * (qy * qz - q0 * qx)], 2)
    r2 = jnp.concatenate([2.0 * (qz * qx - q0 * qy),
                          2.0 * (qz * qy + q0 * qx),
                          q0 * q0 - qx * qx - qy * qy + qz * qz], 2)
    rmat = jnp.concatenate([r0, r1, r2], 1)
    r_ref[...] = rmat

    rmux = rmat[:, :, 0:1] * mux_c[:, 0:1, :]
    for bidx in range(1, 3):
        rmux = rmux + rmat[:, :, bidx:bidx + 1] * mux_c[:, bidx:bidx + 1, :]
    t_ref[...] = muy_c - rmux


def kernel(src_embedding, tgt_embedding, src, tgt, temperature, is_corr):
    B, d, N = src_embedding.shape
    ti = _TI
    ni = N // ti
    cpb = _NW // B
    temp3 = temperature.reshape(B, 1, 1)

    src_t = jnp.transpose(src, (0, 2, 1))
    xp = jnp.repeat(
        jnp.concatenate([jnp.ones((B, N, 1), jnp.float32), src_t], axis=2),
        4, axis=2)
    weight4, corres4, u16 = pl.pallas_call(
        _stats_kernel,
        grid=(B, ni),
        in_specs=[
            pl.BlockSpec((1, d, ti), lambda b, i: (b, 0, i)),
            pl.BlockSpec((1, d, N), lambda b, i: (b, 0, 0)),
            pl.BlockSpec((1, ti, 16), lambda b, i: (b, i, 0)),
            pl.BlockSpec((1, 1, 1), lambda b, i: (b, 0, 0)),
        ],
        out_specs=[
            pl.BlockSpec((1, 1, ti, 1), lambda b, i: (b, i, 0, 0)),
            pl.BlockSpec((1, 1, ti, 1), lambda b, i: (b, i, 0, 0)),
            pl.BlockSpec((1, 1, ti, 16), lambda b, i: (b, i, 0, 0)),
        ],
        out_shape=[
            jax.ShapeDtypeStruct((B, ni, ti, 1), jnp.float32),
            jax.ShapeDtypeStruct((B, ni, ti, 1), jnp.int32),
            jax.ShapeDtypeStruct((B, ni, ti, 16), jnp.float32),
        ],
    )(src_embedding, tgt_embedding, xp, temp3)

    weight2 = weight4.reshape(B, N)
    corres2 = corres4.reshape(B, N)

    tgt_t = jnp.transpose(tgt, (0, 2, 1))
    v4 = jnp.concatenate([jnp.ones((B, N, 1), jnp.float32), tgt_t], axis=2)
    tgt128 = jnp.pad(jnp.tile(v4, (1, 1, 4)), ((0, 0), (0, 0), (0, 112)))

    parts = _make_sc_reduce(B, N)(tgt128, corres2, u16.reshape(B, N, 16))
    parts3 = parts.reshape(B, cpb, 16)

    rmat, tvec = pl.pallas_call(
        _solve_kernel,
        out_shape=[
            jax.ShapeDtypeStruct((B, 3, 3), jnp.float32),
            jax.ShapeDtypeStruct((B, 3, 1), jnp.float32),
        ],
    )(parts3)

    return (rmat, tvec.reshape(B, 3),
            corres2.reshape(B, N, 1), weight2.reshape(B, N, 1))


# speedup vs baseline: 1.1909x; 1.0140x over previous
import functools
import math

import jax
import jax.numpy as jnp
from jax import lax
from jax.experimental import pallas as pl
from jax.experimental.pallas import tpu as pltpu
from jax.experimental.pallas import tpu_sc as plsc

_TI = 2048
_NC = 2
_NS = 16
_NW = _NC * _NS

_PAIRS = [(0, 1), (0, 2), (0, 3), (1, 2), (1, 3), (2, 3)]
_N_SWEEPS = 5


_NCHUNK = 2


def _stats_kernel(se_ref, te_ref, xp_ref, temp_ref, w_ref, c_ref, u_ref):
    ti = _TI
    n = te_ref.shape[2]
    nc = n // _NCHUNK
    se = se_ref[0]
    sqd = math.sqrt(se_ref.shape[1])
    temp = temp_ref[0, 0, 0]

    m = None
    cor = None
    ssum = None
    for ci in range(_NCHUNK):
        te = te_ref[0, :, pl.ds(ci * nc, nc)]
        s = lax.dot_general(se, te, (((0,), (0,)), ((), ())),
                            preferred_element_type=jnp.float32)
        sc = temp * (s / sqd)
        cm = jnp.max(sc, axis=1, keepdims=True)
        iota = lax.broadcasted_iota(jnp.int32, (ti, nc), 1) + ci * nc
        ca = jnp.min(jnp.where(sc == cm, iota, n), axis=1, keepdims=True)
        if ci == 0:
            m = cm
            cor = ca
            ssum = jnp.sum(jnp.exp(sc - m), axis=1, keepdims=True)
        else:
            m_new = jnp.maximum(m, cm)
            cor = jnp.where(cm > m, ca, cor)
            ssum = (ssum * jnp.exp(m - m_new)
                    + jnp.sum(jnp.exp(sc - m_new), axis=1, keepdims=True))
            m = m_new
    w = 1.0 / ssum
    w_ref[0, 0] = w
    c_ref[0, 0] = cor
    u_ref[0, 0] = w * xp_ref[0]


def _make_sc_reduce(B, N):
    cpb = _NW // B
    rpw = N // cpb
    steps = rpw // 16

    mesh = plsc.VectorSubcoreMesh(core_axis_name="c", subcore_axis_name="s",
                                  num_cores=_NC, num_subcores=_NS)

    @functools.partial(
        pl.kernel, mesh=mesh,
        out_type=jax.ShapeDtypeStruct((_NW, 1, 16), jnp.float32),
        scratch_types=[
            pltpu.VMEM((rpw,), jnp.int32),
            pltpu.VMEM((rpw, 16), jnp.float32),
            pltpu.VMEM((128, 128), jnp.float32),
            pltpu.VMEM((128, 128), jnp.float32),
            pltpu.VMEM((1, 16), jnp.float32),
            pltpu.SemaphoreType.DMA,
            pltpu.SemaphoreType.DMA,
        ],
    )
    def sc_reduce(tgt16_hbm, cor_hbm, u_hbm, out_hbm,
                  idx_v, uv, yv0, yv1, outv, sem0, sem1):
        wid = lax.axis_index("s") * _NC + lax.axis_index("c")
        b = wid // cpb
        base = (wid % cpb) * rpw
        pltpu.sync_copy(cor_hbm.at[b, pl.ds(base, rpw)], idx_v)
        pltpu.sync_copy(u_hbm.at[b, pl.ds(base, rpw)], uv)

        nch = rpw // 128
        bufs = [yv0, yv1]
        sems = [sem0, sem1]

        def fire(ch):
            return pltpu.async_copy(
                tgt16_hbm.at[b].at[idx_v.at[pl.ds(ch * 128, 128)]],
                bufs[ch % 2], sems[ch % 2])

        accs = [jnp.zeros((16,), jnp.float32) for _ in range(4)]
        pending = fire(0)
        for ch in range(nch):
            nxt = fire(ch + 1) if ch + 1 < nch else None
            pending.wait()
            yv = bufs[ch % 2]
            for r in range(0, 128, 4):
                for j in range(4):
                    accs[j] = (accs[j] + uv[ch * 128 + r + j, :]
                               * yv[r + j, pl.ds(0, 16)])
            pending = nxt
        outv[0, :] = (accs[0] + accs[1]) + (accs[2] + accs[3])
        pltpu.sync_copy(outv, out_hbm.at[wid])

    return sc_reduce


def _solve_kernel(parts_ref, r_ref, t_ref):
    eps = 1e-07
    pv = parts_ref[...]
    vsum = jnp.sum(pv, axis=1, keepdims=True)

    def e(j):
        return vsum[:, :, j:j + 1]

    sw = e(0)
    den = sw + eps
    c = sw / den
    mux_c = jnp.concatenate([e(4), e(8), e(12)], 1) / den
    muy_r = jnp.concatenate([e(1), e(2), e(3)], 2) / den
    muy_c = jnp.concatenate([e(1), e(2), e(3)], 1) / den
    rows = [jnp.concatenate([e(4 * i + 5), e(4 * i + 6), e(4 * i + 7)], 2)
            for i in range(3)]
    araw = jnp.concatenate(rows, 1)
    a = araw / den - (2.0 - c) * (mux_c * muy_r)

    def ent(i, j):
        return a[:, i:i + 1, j:j + 1]

    s00, s01, s02 = ent(0, 0), ent(0, 1), ent(0, 2)
    s10, s11, s12 = ent(1, 0), ent(1, 1), ent(1, 2)
    s20, s21, s22 = ent(2, 0), ent(2, 1), ent(2, 2)

    km = {}
    km[(0, 0)] = s00 + s11 + s22
    km[(0, 1)] = s12 - s21
    km[(0, 2)] = s20 - s02
    km[(0, 3)] = s01 - s10
    km[(1, 1)] = s00 - s11 - s22
    km[(1, 2)] = s01 + s10
    km[(1, 3)] = s20 + s02
    km[(2, 2)] = -s00 + s11 - s22
    km[(2, 3)] = s12 + s21
    km[(3, 3)] = -s00 - s11 + s22

    def kget(i, j):
        return km[(i, j)] if i <= j else km[(j, i)]

    def kset(i, j, val):
        km[(min(i, j), max(i, j))] = val

    one = jnp.ones_like(s00)
    zero = jnp.zeros_like(s00)
    vm = {(i, j): (one if i == j else zero)
          for i in range(4) for j in range(4)}

    for _ in range(_N_SWEEPS):
        for (p, q) in _PAIRS:
            apq = kget(p, q)
            diff = kget(q, q) - kget(p, p)
            sn = jnp.where(diff >= 0.0, 1.0, -1.0)
            den = jnp.abs(diff) + jnp.sqrt(diff * diff + 4.0 * apq * apq) + 1e-38
            tj = 2.0 * sn * apq / den
            cc = lax.rsqrt(tj * tj + 1.0)
            ss = tj * cc
            kpp, kqq, kpq = kget(p, p), kget(q, q), kget(p, q)
            kset(p, p, cc * cc * kpp - 2.0 * ss * cc * kpq + ss * ss * kqq)
            kset(q, q, ss * ss * kpp + 2.0 * ss * cc * kpq + cc * cc * kqq)
            kset(p, q, zero)
            for r in range(4):
                if r == p or r == q:
                    continue
                kpr, kqr = kget(p, r), kget(q, r)
                kset(p, r, cc * kpr - ss * kqr)
                kset(q, r, ss * kpr + cc * kqr)
            for i in range(4):
                vip, viq = vm[(i, p)], vm[(i, q)]
                vm[(i, p)] = cc * vip - ss * viq
                vm[(i, q)] = ss * vip + cc * viq

    d0, d1, d2, d3 = kget(0, 0), kget(1, 1), kget(2, 2), kget(3, 3)
    dmax = jnp.maximum(jnp.maximum(d0, d1), jnp.maximum(d2, d3))

    def pick(i):
        return jnp.where(d0 == dmax, vm[(i, 0)],
                         jnp.where(d1 == dmax, vm[(i, 1)],
                                   jnp.where(d2 == dmax, vm[(i, 2)],
                                             vm[(i, 3)])))

    q0, qx, qy, qz = pick(0), pick(1), pick(2), pick(3)
    qn = 1.0 / jnp.sqrt(q0 * q0 + qx * qx + qy * qy + qz * qz)
    q0, qx, qy, qz = q0 * qn, qx * qn, qy * qn, qz * qn
    r0 = jnp.concatenate([q0 * q0 + qx * qx - qy * qy - qz * qz,
                          2.0 * (qx * qy - q0 * qz),
                          2.0 * (qx * qz + q0 * qy)], 2)
    r1 = jnp.concatenate([2.0 * (qy * qx + q0 * qz),
                          q0 * q0 - qx * qx + qy * qy - qz * qz,
                          2.0 * (qy * qz - q0 * qx)], 2)
    r2 = jnp.concatenate([2.0 * (qz * qx - q0 * qy),
                          2.0 * (qz * qy + q0 * qx),
                          q0 * q0 - qx * qx - qy * qy + qz * qz], 2)
    rmat = jnp.concatenate([r0, r1, r2], 1)
    r_ref[...] = rmat

    rmux = rmat[:, :, 0:1] * mux_c[:, 0:1, :]
    for bidx in range(1, 3):
        rmux = rmux + rmat[:, :, bidx:bidx + 1] * mux_c[:, bidx:bidx + 1, :]
    t_ref[...] = muy_c - rmux


def kernel(src_embedding, tgt_embedding, src, tgt, temperature, is_corr):
    B, d, N = src_embedding.shape
    ti = _TI
    ni = N // ti
    cpb = _NW // B
    temp3 = temperature.reshape(B, 1, 1)

    src_t = jnp.transpose(src, (0, 2, 1))
    xp = jnp.repeat(
        jnp.concatenate([jnp.ones((B, N, 1), jnp.float32), src_t], axis=2),
        4, axis=2)
    weight4, corres4, u16 = pl.pallas_call(
        _stats_kernel,
        grid=(B, ni),
        in_specs=[
            pl.BlockSpec((1, d, ti), lambda b, i: (b, 0, i)),
            pl.BlockSpec((1, d, N), lambda b, i: (b, 0, 0)),
            pl.BlockSpec((1, ti, 16), lambda b, i: (b, i, 0)),
            pl.BlockSpec((1, 1, 1), lambda b, i: (b, 0, 0)),
        ],
        out_specs=[
            pl.BlockSpec((1, 1, ti, 1), lambda b, i: (b, i, 0, 0)),
            pl.BlockSpec((1, 1, ti, 1), lambda b, i: (b, i, 0, 0)),
            pl.BlockSpec((1, 1, ti, 16), lambda b, i: (b, i, 0, 0)),
        ],
        out_shape=[
            jax.ShapeDtypeStruct((B, ni, ti, 1), jnp.float32),
            jax.ShapeDtypeStruct((B, ni, ti, 1), jnp.int32),
            jax.ShapeDtypeStruct((B, ni, ti, 16), jnp.float32),
        ],
    )(src_embedding, tgt_embedding, xp, temp3)

    weight2 = weight4.reshape(B, N)
    corres2 = corres4.reshape(B, N)

    tgt_t = jnp.transpose(tgt, (0, 2, 1))
    v4 = jnp.concatenate([jnp.ones((B, N, 1), jnp.float32), tgt_t], axis=2)
    tgt128 = jnp.pad(jnp.tile(v4, (1, 1, 4)), ((0, 0), (0, 0), (0, 112)))

    parts = _make_sc_reduce(B, N)(tgt128, corres2, u16.reshape(B, N, 16))
    parts3 = parts.reshape(B, cpb, 16)

    rmat, tvec = pl.pallas_call(
        _solve_kernel,
        out_shape=[
            jax.ShapeDtypeStruct((B, 3, 3), jnp.float32),
            jax.ShapeDtypeStruct((B, 3, 1), jnp.float32),
        ],
    )(parts3)

    return (rmat, tvec.reshape(B, 3),
            corres2.reshape(B, N, 1), weight2.reshape(B, N, 1))
